# baseline (speedup 1.0000x reference)
import functools

import jax
import jax.numpy as jnp
from jax import lax
from jax.experimental import pallas as pl
from jax.experimental.pallas import tpu as pltpu
from jax.experimental.pallas import tpu_sc as plsc

N = 10000
E = 320000
D = 128
OUT = 128

NC = 2
NS = 16
NW = NC * NS
EPT = E // NW
K = 80
EPTP = 10240
NCHUNK = EPTP // K
PAD = EPTP - EPT
NP = 10240
RPT = NP // NS
SROWS = 80
DEN_SZ = 10016


def _t1_body(x_ref, w_ref, b_ref, a_ref, wh_ref, s_ref, c_ref):
    wh = jnp.dot(x_ref[:], w_ref[:], preferred_element_type=jnp.float32)
    wh = wh + b_ref[:]
    wh_ref[:] = wh
    s = lax.dot_general(a_ref[:], wh, (((1,), (1,)), ((), ())),
                        preferred_element_type=jnp.float32)
    spad = jnp.concatenate(
        [s, jnp.zeros((2, SROWS * 128 - N), jnp.float32)], axis=1)
    s_ref[:] = spad.reshape(2, SROWS, 128)
    t = jnp.max(s[0]) + jnp.max(s[1])
    c = jnp.where(t >= 0, t, t * 0.01)
    c_ref[:] = jnp.full((1, 128), c, dtype=jnp.float32)


def _t3_body(g_ref, den_ref, h_ref):
    num = g_ref[0] + g_ref[1]
    den = jnp.sum(den_ref[0], axis=0)
    den = den[:, None]
    h_ref[:] = jnp.where(den > 0, num / den, 0.0)


def _sc_body(idx_hbm, s_hbm, c_hbm, wh_hbm, zeros_hbm,
             g_out, den_out,
             idxring, rows2, ex_v, den_v, c_v,
             idxsem, gsem, ssem, g_s):
    cid = lax.axis_index("c")
    sid = lax.axis_index("s")
    wid = cid * NS + sid

    pltpu.sync_copy(c_hbm, c_v)
    cvec = c_v[pl.ds(0, 16)]

    pltpu.sync_copy(zeros_hbm, g_s.at[pl.ds(sid * RPT, RPT)])

    def zero_body(i, carry):
        den_v[pl.ds(i * 16, 16)] = jnp.zeros((16,), jnp.float32)
        return carry

    lax.fori_loop(0, DEN_SZ // 16, zero_body, 0)
    plsc.subcore_barrier()

    lanes = lax.iota(jnp.int32, 16)
    zeros16 = jnp.zeros((16,), jnp.int32)
    ones16 = jnp.ones((16,), jnp.int32)
    c127 = jnp.full((16,), 127, jnp.int32)

    pltpu.sync_copy(s_hbm, rows2)

    for u in range(4):
        pltpu.async_copy(idx_hbm.at[wid, u], idxring.at[u], idxsem.at[u])

    def p1_chunk(c, slot):
        pltpu.make_async_copy(idx_hbm.at[wid, c],
                              idxring.at[slot], idxsem.at[slot]).wait()
        for g in range(K // 16):
            src16 = idxring[slot, 0, pl.ds(g * 16, 16)]
            dst16 = idxring[slot, 1, pl.ds(g * 16, 16)]
            sq = lax.shift_right_logical(src16, 7)
            sr = lax.bitwise_and(src16, c127)
            dq = lax.shift_right_logical(dst16, 7)
            dr = lax.bitwise_and(dst16, c127)
            v1 = plsc.load_gather(rows2, [zeros16, sq, sr])
            v2 = plsc.load_gather(rows2, [ones16, dq, dr])
            t = v1 + v2
            e = jnp.where(t >= 0, t, t * 0.01)
            ex = jnp.exp(e - cvec)
            ex_v[pl.ds(c * K + g * 16, 16)] = ex
            for j in range(16):
                plsc.addupdate_scatter(den_v, [dst16], ex, mask=lanes == j)
        @pl.when(c + 4 < NCHUNK)
        def _():
            pltpu.async_copy(idx_hbm.at[wid, c + 4],
                             idxring.at[slot], idxsem.at[slot])

    def p1_body(i, carry):
        for u in range(4):
            p1_chunk(i * 4 + u, u)
        return carry

    lax.fori_loop(0, NCHUNK // 4, p1_body, 0)

    for u in range(4):
        pltpu.async_copy(idx_hbm.at[wid, u], idxring.at[u], idxsem.at[u])
    pltpu.make_async_copy(idx_hbm.at[wid, 0],
                          idxring.at[0], idxsem.at[0]).wait()
    pltpu.async_copy(wh_hbm.at[idxring.at[0, 0]], rows2.at[0], gsem.at[0])

    def p2_chunk(c, slot):
        b = slot & 1
        pltpu.make_async_copy(wh_hbm.at[idxring.at[slot, 0]],
                              rows2.at[b], gsem.at[b]).wait()
        @pl.when(c + 1 < NCHUNK)
        def _():
            nslot = (slot + 1) & 3
            nb = (slot + 1) & 1

            @pl.when(c >= 1)
            def _():
                pltpu.make_async_copy(rows2.at[nb],
                                      g_s.at[idxring.at[nslot, 1]],
                                      ssem.at[nb]).wait()
            pltpu.make_async_copy(idx_hbm.at[wid, c + 1],
                                  idxring.at[nslot], idxsem.at[nslot]).wait()
            pltpu.async_copy(wh_hbm.at[idxring.at[nslot, 0]],
                             rows2.at[nb], gsem.at[nb])
        for g in range(K // 16):
            exg = ex_v[pl.ds(c * K + g * 16, 16)]
            for j in range(16):
                r = g * 16 + j
                bj = lax.gather(
                    exg, jnp.full((16, 1), j, jnp.int32),
                    lax.GatherDimensionNumbers(
                        offset_dims=(), collapsed_slice_dims=(0,),
                        start_index_map=(0,)),
                    (1,),
                    mode=lax.GatherScatterMode.PROMISE_IN_BOUNDS)
                for q in range(D // 16):
                    rows2[b, r, pl.ds(q * 16, 16)] = (
                        rows2[b, r, pl.ds(q * 16, 16)] * bj)
        pltpu.async_copy(rows2.at[b], g_s.at[idxring.at[slot, 1]],
                         ssem.at[b], add=True)
        @pl.when((c >= 1) & (c + 3 < NCHUNK))
        def _():
            pslot = (slot + 3) & 3
            pltpu.async_copy(idx_hbm.at[wid, c + 3],
                             idxring.at[pslot], idxsem.at[pslot])

    def p2_body(i, carry):
        for u in range(4):
            p2_chunk(i * 4 + u, u)
        return carry

    lax.fori_loop(0, NCHUNK // 4, p2_body, 0)
    pltpu.make_async_copy(rows2.at[0], g_s.at[idxring.at[2, 1]],
                          ssem.at[0]).wait()
    pltpu.make_async_copy(rows2.at[1], g_s.at[idxring.at[3, 1]],
                          ssem.at[1]).wait()
    plsc.subcore_barrier()

    row0 = sid * RPT
    pltpu.sync_copy(g_s.at[pl.ds(row0, RPT)],
                    g_out.at[cid, pl.ds(row0, RPT)])
    pltpu.sync_copy(den_v, den_out.at[cid, sid])


_sc_mesh = plsc.VectorSubcoreMesh(core_axis_name="c", subcore_axis_name="s",
                                  num_cores=NC, num_subcores=NS)

_sc_kernel = functools.partial(
    pl.kernel,
    out_type=(jax.ShapeDtypeStruct((NC, NP, D), jnp.float32),
              jax.ShapeDtypeStruct((NC, NS, DEN_SZ), jnp.float32)),
    mesh=_sc_mesh,
    scratch_types=[
        pltpu.VMEM((4, 2, K), jnp.int32),
        pltpu.VMEM((2, K, D), jnp.float32),
        pltpu.VMEM((EPTP,), jnp.float32),
        pltpu.VMEM((DEN_SZ,), jnp.float32),
        pltpu.VMEM((128,), jnp.float32),
        pltpu.SemaphoreType.DMA((4,)),
        pltpu.SemaphoreType.DMA((2,)),
        pltpu.SemaphoreType.DMA((2,)),
        pltpu.VMEM_SHARED((NP, D), jnp.float32),
    ],
    compiler_params=pltpu.CompilerParams(needs_layout_passes=False),
)(_sc_body)


@jax.jit
def kernel(x, edge_index, W, b, a_w):
    src2 = edge_index[0].reshape(NW, EPT)
    dst2 = edge_index[1].reshape(NW, EPT)
    src2 = jnp.concatenate(
        [src2, jnp.zeros((NW, PAD), jnp.int32)], axis=1)
    dst2 = jnp.concatenate(
        [dst2, jnp.full((NW, PAD), N, jnp.int32)], axis=1)
    idx = jnp.stack([src2.reshape(NW, NCHUNK, K),
                     dst2.reshape(NW, NCHUNK, K)], axis=2)

    a2 = a_w.reshape(2, D)
    b2 = b.reshape(1, OUT)

    wh, s, c = pl.pallas_call(
        _t1_body,
        out_shape=[
            jax.ShapeDtypeStruct((N, OUT), jnp.float32),
            jax.ShapeDtypeStruct((2, SROWS, 128), jnp.float32),
            jax.ShapeDtypeStruct((1, 128), jnp.float32),
        ],
    )(x, W, b2, a2)

    c128 = c.reshape(128)
    zeros = jnp.zeros((RPT, D), dtype=jnp.float32)

    g, den = _sc_kernel(idx, s, c128, wh, zeros)

    BR = 1000
    h = pl.pallas_call(
        _t3_body,
        grid=(N // BR,),
        in_specs=[
            pl.BlockSpec((NC, BR, D), lambda i: (0, i, 0)),
            pl.BlockSpec((1, NC * NS, BR), lambda i: (i, 0, 0)),
        ],
        out_specs=pl.BlockSpec((BR, OUT), lambda i: (i, 0)),
        out_shape=jax.ShapeDtypeStruct((N, OUT), jnp.float32),
    )(g, den[:, :, :N].reshape(NC * NS, N // BR, BR).transpose(1, 0, 2))
    return h


# speedup vs baseline: 1.6216x; 1.6216x over previous
import functools

import jax
import jax.numpy as jnp
from jax import lax
from jax.experimental import pallas as pl
from jax.experimental.pallas import tpu as pltpu
from jax.experimental.pallas import tpu_sc as plsc

N = 10000
E = 320000
D = 128
OUT = 128

NC = 2
NS = 16
NW = NC * NS
EPT = E // NW
K = 80
SB = 25
NSUPER = EPT // (K * SB)
NP = 10240
RPT = NP // NS


def _t1_body(x_ref, w_ref, b_ref, a_ref, wh_ref, s_ref, c_ref):
    wh = jnp.dot(x_ref[:], w_ref[:], preferred_element_type=jnp.float32)
    wh = wh + b_ref[:]
    wh_ref[:] = wh
    s = lax.dot_general(a_ref[:], wh, (((1,), (1,)), ((), ())),
                        preferred_element_type=jnp.float32)
    s_ref[:] = s
    t = jnp.max(s[0]) + jnp.max(s[1])
    c = jnp.where(t >= 0, t, t * 0.01)
    c_ref[:] = jnp.full((1, 128), c, dtype=jnp.float32)


def _t3_body(g_ref, den_ref, h_ref):
    num = g_ref[0] + g_ref[1]
    den = jnp.sum(den_ref[0], axis=0)
    den = den[:, None]
    h_ref[:] = jnp.where(den > 0, num / den, 0.0)


def _sc_body(src4, dst4, s1_hbm, s2_hbm, c_hbm, wh_hbm, zeros_hbm,
             g_out, den_out,
             src2d_v, dst2d_v, s1_v, s2_v, c_v,
             rows_v, den_v, g_s):
    cid = lax.axis_index("c")
    sid = lax.axis_index("s")
    wid = cid * NS + sid

    pltpu.sync_copy(s1_hbm, s1_v)
    pltpu.sync_copy(s2_hbm, s2_v)
    pltpu.sync_copy(c_hbm, c_v)
    cvec = c_v[pl.ds(0, 16)]

    pltpu.sync_copy(zeros_hbm, g_s.at[pl.ds(sid * RPT, RPT)])

    def zero_body(i, carry):
        den_v[pl.ds(i * 16, 16)] = jnp.zeros((16,), jnp.float32)
        return carry

    lax.fori_loop(0, N // 16, zero_body, 0)
    plsc.subcore_barrier()

    lanes = lax.iota(jnp.int32, 16)

    def super_body(si, carry):
        pltpu.sync_copy(src4.at[wid, si], src2d_v)
        pltpu.sync_copy(dst4.at[wid, si], dst2d_v)

        def chunk_body(cj, carry2):
            pltpu.sync_copy(wh_hbm.at[src2d_v.at[cj]], rows_v)
            for g in range(K // 16):
                src16 = src2d_v[cj, pl.ds(g * 16, 16)]
                dst16 = dst2d_v[cj, pl.ds(g * 16, 16)]
                v1 = plsc.load_gather(s1_v, [src16])
                v2 = plsc.load_gather(s2_v, [dst16])
                t = v1 + v2
                e = jnp.where(t >= 0, t, t * 0.01)
                ex = jnp.exp(e - cvec)
                bjs = [lax.gather(
                    ex, jnp.full((16, 1), j, jnp.int32),
                    lax.GatherDimensionNumbers(
                        offset_dims=(), collapsed_slice_dims=(0,),
                        start_index_map=(0,)),
                    (1,),
                    mode=lax.GatherScatterMode.PROMISE_IN_BOUNDS)
                    for j in range(16)]
                for j in range(16):
                    plsc.addupdate_scatter(den_v, [dst16], ex,
                                           mask=lanes == j)
                for j in range(16):
                    r = g * 16 + j
                    for q in range(D // 16):
                        rows_v[r, pl.ds(q * 16, 16)] = (
                            rows_v[r, pl.ds(q * 16, 16)] * bjs[j])
            pltpu.sync_copy(rows_v, g_s.at[dst2d_v.at[cj]], add=True)
            return carry2

        lax.fori_loop(0, SB, chunk_body, 0)
        return carry

    lax.fori_loop(0, NSUPER, super_body, 0)
    plsc.subcore_barrier()

    row0 = sid * RPT
    pltpu.sync_copy(g_s.at[pl.ds(row0, RPT)],
                    g_out.at[cid, pl.ds(row0, RPT)])
    pltpu.sync_copy(den_v, den_out.at[cid, sid])


_sc_mesh = plsc.VectorSubcoreMesh(core_axis_name="c", subcore_axis_name="s",
                                  num_cores=NC, num_subcores=NS)

_sc_kernel = functools.partial(
    pl.kernel,
    out_type=(jax.ShapeDtypeStruct((NC, NP, D), jnp.float32),
              jax.ShapeDtypeStruct((NC, NS, N), jnp.float32)),
    mesh=_sc_mesh,
    scratch_types=[
        pltpu.VMEM((SB, K), jnp.int32),
        pltpu.VMEM((SB, K), jnp.int32),
        pltpu.VMEM((N,), jnp.float32),
        pltpu.VMEM((N,), jnp.float32),
        pltpu.VMEM((128,), jnp.float32),
        pltpu.VMEM((K, D), jnp.float32),
        pltpu.VMEM((N,), jnp.float32),
        pltpu.VMEM_SHARED((NP, D), jnp.float32),
    ],
    compiler_params=pltpu.CompilerParams(needs_layout_passes=False),
)(_sc_body)


@jax.jit
def kernel(x, edge_index, W, b, a_w):
    src4 = edge_index[0].reshape(NW, NSUPER, SB, K)
    dst4 = edge_index[1].reshape(NW, NSUPER, SB, K)
    a2 = a_w.reshape(2, D)
    b2 = b.reshape(1, OUT)

    wh, s, c = pl.pallas_call(
        _t1_body,
        out_shape=[
            jax.ShapeDtypeStruct((N, OUT), jnp.float32),
            jax.ShapeDtypeStruct((2, N), jnp.float32),
            jax.ShapeDtypeStruct((1, 128), jnp.float32),
        ],
    )(x, W, b2, a2)

    s1 = s[0]
    s2 = s[1]
    c128 = c.reshape(128)
    zeros = jnp.zeros((RPT, D), dtype=jnp.float32)

    g, den = _sc_kernel(src4, dst4, s1, s2, c128, wh, zeros)

    BR = 1000
    h = pl.pallas_call(
        _t3_body,
        grid=(N // BR,),
        in_specs=[
            pl.BlockSpec((NC, BR, D), lambda i: (0, i, 0)),
            pl.BlockSpec((1, NC * NS, BR), lambda i: (i, 0, 0)),
        ],
        out_specs=pl.BlockSpec((BR, OUT), lambda i: (i, 0)),
        out_shape=jax.ShapeDtypeStruct((N, OUT), jnp.float32),
    )(g, den.reshape(NC * NS, N // BR, BR).transpose(1, 0, 2))
    return h


# speedup vs baseline: 1.7244x; 1.0634x over previous
import functools

import jax
import jax.numpy as jnp
from jax import lax
from jax.experimental import pallas as pl
from jax.experimental.pallas import tpu as pltpu
from jax.experimental.pallas import tpu_sc as plsc

N = 10000
E = 320000
D = 128
OUT = 128

NC = 2
NS = 16
NW = NC * NS
EPT = E // NW
K = 80
SB = 25
NSUPER = EPT // (K * SB)
NP = 10240
RPT = NP // NS


def _t1_body(x_ref, w_ref, b_ref, a_ref, wh_ref, s_ref, c_ref):
    wh = jnp.dot(x_ref[:], w_ref[:], preferred_element_type=jnp.float32)
    wh = wh + b_ref[:]
    wh_ref[:] = wh
    s = lax.dot_general(a_ref[:], wh, (((1,), (1,)), ((), ())),
                        preferred_element_type=jnp.float32)
    s_ref[:] = s
    t = jnp.max(s[0]) + jnp.max(s[1])
    c = jnp.where(t >= 0, t, t * 0.01)
    c_ref[:] = jnp.full((1, 128), c, dtype=jnp.float32)


def _t3_body(g_ref, den_ref, h_ref):
    num = g_ref[0] + g_ref[1]
    den = jnp.sum(den_ref[0], axis=0)
    den = den[:, None]
    h_ref[:] = jnp.where(den > 0, num / den, 0.0)


def _sc_body(src4, dst4, s1_hbm, s2_hbm, c_hbm, wh_hbm, zeros_hbm,
             g_out, den_out,
             src2d_v, dst2d_v, s1_v, s2_v, c_v,
             rows_v, den_v, gsem, g_s):
    cid = lax.axis_index("c")
    sid = lax.axis_index("s")
    wid = cid * NS + sid

    pltpu.sync_copy(s1_hbm, s1_v)
    pltpu.sync_copy(s2_hbm, s2_v)
    pltpu.sync_copy(c_hbm, c_v)
    cvec = c_v[pl.ds(0, 16)]

    pltpu.sync_copy(zeros_hbm, g_s.at[pl.ds(sid * RPT, RPT)])

    def zero_body(i, carry):
        den_v[pl.ds(i * 16, 16)] = jnp.zeros((16,), jnp.float32)
        return carry

    lax.fori_loop(0, N // 16, zero_body, 0)
    plsc.subcore_barrier()

    lanes = lax.iota(jnp.int32, 16)

    def super_body(si, carry):
        pltpu.sync_copy(src4.at[wid, si], src2d_v)
        pltpu.sync_copy(dst4.at[wid, si], dst2d_v)

        def chunk_body(cj, carry2):
            gdesc = pltpu.async_copy(wh_hbm.at[src2d_v.at[cj]],
                                     rows_v, gsem)
            exs = []
            for g in range(K // 16):
                src16 = src2d_v[cj, pl.ds(g * 16, 16)]
                dst16 = dst2d_v[cj, pl.ds(g * 16, 16)]
                v1 = plsc.load_gather(s1_v, [src16])
                v2 = plsc.load_gather(s2_v, [dst16])
                t = v1 + v2
                e = jnp.where(t >= 0, t, t * 0.01)
                ex = jnp.exp(e - cvec)
                for j in range(16):
                    plsc.addupdate_scatter(den_v, [dst16], ex,
                                           mask=lanes == j)
                exs.append(ex)
            gdesc.wait()
            for g in range(K // 16):
                ex = exs[g]
                bjs = [lax.gather(
                    ex, jnp.full((16, 1), j, jnp.int32),
                    lax.GatherDimensionNumbers(
                        offset_dims=(), collapsed_slice_dims=(0,),
                        start_index_map=(0,)),
                    (1,),
                    mode=lax.GatherScatterMode.PROMISE_IN_BOUNDS)
                    for j in range(16)]
                for j in range(16):
                    r = g * 16 + j
                    for q in range(D // 16):
                        rows_v[r, pl.ds(q * 16, 16)] = (
                            rows_v[r, pl.ds(q * 16, 16)] * bjs[j])
            pltpu.sync_copy(rows_v, g_s.at[dst2d_v.at[cj]], add=True)
            return carry2

        lax.fori_loop(0, SB, chunk_body, 0)
        return carry

    lax.fori_loop(0, NSUPER, super_body, 0)
    plsc.subcore_barrier()

    row0 = sid * RPT
    pltpu.sync_copy(g_s.at[pl.ds(row0, RPT)],
                    g_out.at[cid, pl.ds(row0, RPT)])
    pltpu.sync_copy(den_v, den_out.at[cid, sid])


_sc_mesh = plsc.VectorSubcoreMesh(core_axis_name="c", subcore_axis_name="s",
                                  num_cores=NC, num_subcores=NS)

_sc_kernel = functools.partial(
    pl.kernel,
    out_type=(jax.ShapeDtypeStruct((NC, NP, D), jnp.float32),
              jax.ShapeDtypeStruct((NC, NS, N), jnp.float32)),
    mesh=_sc_mesh,
    scratch_types=[
        pltpu.VMEM((SB, K), jnp.int32),
        pltpu.VMEM((SB, K), jnp.int32),
        pltpu.VMEM((N,), jnp.float32),
        pltpu.VMEM((N,), jnp.float32),
        pltpu.VMEM((128,), jnp.float32),
        pltpu.VMEM((K, D), jnp.float32),
        pltpu.VMEM((N,), jnp.float32),
        pltpu.SemaphoreType.DMA,
        pltpu.VMEM_SHARED((NP, D), jnp.float32),
    ],
    compiler_params=pltpu.CompilerParams(needs_layout_passes=False),
)(_sc_body)


@jax.jit
def kernel(x, edge_index, W, b, a_w):
    src4 = edge_index[0].reshape(NW, NSUPER, SB, K)
    dst4 = edge_index[1].reshape(NW, NSUPER, SB, K)
    a2 = a_w.reshape(2, D)
    b2 = b.reshape(1, OUT)

    wh, s, c = pl.pallas_call(
        _t1_body,
        out_shape=[
            jax.ShapeDtypeStruct((N, OUT), jnp.float32),
            jax.ShapeDtypeStruct((2, N), jnp.float32),
            jax.ShapeDtypeStruct((1, 128), jnp.float32),
        ],
    )(x, W, b2, a2)

    s1 = s[0]
    s2 = s[1]
    c128 = c.reshape(128)
    zeros = jnp.zeros((RPT, D), dtype=jnp.float32)

    g, den = _sc_kernel(src4, dst4, s1, s2, c128, wh, zeros)

    BR = 1000
    h = pl.pallas_call(
        _t3_body,
        grid=(N // BR,),
        in_specs=[
            pl.BlockSpec((NC, BR, D), lambda i: (0, i, 0)),
            pl.BlockSpec((1, NC * NS, BR), lambda i: (i, 0, 0)),
        ],
        out_specs=pl.BlockSpec((BR, OUT), lambda i: (i, 0)),
        out_shape=jax.ShapeDtypeStruct((N, OUT), jnp.float32),
    )(g, den.reshape(NC * NS, N // BR, BR).transpose(1, 0, 2))
    return h
